# baseline (device time: 26818 ns/iter reference)
import jax
import jax.numpy as jnp
from jax import lax
from jax.experimental import pallas as pl
from jax.experimental.pallas import tpu as pltpu

N_DEV = 4


def kernel(partial, resid, gamma):
    x = partial.reshape(partial.shape[-2], partial.shape[-1])
    m, n = x.shape
    bs = m // 8
    hs = bs // 2
    gamma2d = gamma.reshape(1, n)
    x = pltpu.with_memory_space_constraint(x, pltpu.MemorySpace.HBM)
    resid = pltpu.with_memory_space_constraint(resid, pltpu.MemorySpace.HBM)
    gamma2d = pltpu.with_memory_space_constraint(gamma2d, pltpu.MemorySpace.HBM)

    def body(x_hbm, resid_hbm, gamma_hbm, out_hbm,
             xv_ref, rv_ref, gv_ref,
             xb_ref, r1_ref, r2_ref, ag_ref,
             send_sems, recv_sems, lsem):
        my = lax.axis_index("i")
        p1 = my ^ 1
        p2 = 3 - my

        def rowA(j):
            return j * bs

        def rowB(j):
            return (4 + j) * bs

        send_offs = (rowA(p2 ^ 1), rowB(p2 ^ 1), rowA(p1), rowB(p2))
        held_offs = (rowA(p2), rowB(p1), rowA(my), rowB(my))
        dx = []
        for k, off in enumerate(send_offs + held_offs):
            d = pltpu.make_async_copy(
                x_hbm.at[pl.ds(off, bs), :],
                xv_ref.at[pl.ds(off, bs), :],
                lsem.at[k],
            )
            d.start()
            dx.append(d)
        cp_r = []
        for k, off in enumerate((rowA(my), rowB(my))):
            d = pltpu.make_async_copy(
                resid_hbm.at[pl.ds(off, bs), :],
                rv_ref.at[pl.ds(off, bs), :],
                lsem.at[8 + k],
            )
            d.start()
            cp_r.append(d)
        cp_g = pltpu.make_async_copy(gamma_hbm, gv_ref, lsem.at[26])
        cp_g.start()

        barrier_sem = pltpu.get_barrier_semaphore()
        for nbr in (p1, p2):
            pl.semaphore_signal(
                barrier_sem, inc=1,
                device_id=(nbr,), device_id_type=pl.DeviceIdType.MESH,
            )
        pl.semaphore_wait(barrier_sem, 2)

        def make(src_ref, dst_ref, off, rows, partner, i):
            return pltpu.make_async_remote_copy(
                src_ref=src_ref.at[pl.ds(off, rows), :],
                dst_ref=dst_ref.at[pl.ds(off, rows), :],
                send_sem=send_sems.at[i],
                recv_sem=recv_sems.at[i],
                device_id=(partner,),
                device_id_type=pl.DeviceIdType.MESH,
            )

        def acc(off):
            r1_ref[pl.ds(off, bs), :] = (
                r1_ref[pl.ds(off, bs), :]
                + xv_ref[pl.ds(off, bs), :].astype(jnp.bfloat16)
            )

        t = {}

        def cast_send(dma, off, tgt, i):
            dma.wait()
            xb_ref[pl.ds(off, bs), :] = (
                xv_ref[pl.ds(off, bs), :].astype(jnp.bfloat16)
            )
            t[i] = make(xb_ref, r1_ref, off, bs, tgt, i)
            t[i].start()

        cast_send(dx[0], rowA(p2 ^ 1), p1, 0)
        cast_send(dx[1], rowB(p2 ^ 1), p2, 2)
        cast_send(dx[2], rowA(p1), p1, 1)
        cast_send(dx[3], rowB(p2), p2, 3)

        t[0].wait_recv()
        dx[4].wait()
        acc(rowA(p2))
        for c in (0, 1):
            t[4 + c] = make(r1_ref, r2_ref, rowA(p2) + c * hs, hs, p2, 4 + c)
            t[4 + c].start()

        t[2].wait_recv()
        dx[5].wait()
        acc(rowB(p1))
        for c in (0, 1):
            t[6 + c] = make(r1_ref, r2_ref, rowB(p1) + c * hs, hs, p1, 6 + c)
            t[6 + c].start()

        t[1].wait_recv()
        dx[6].wait()
        acc(rowA(my))
        t[3].wait_recv()
        dx[7].wait()
        acc(rowB(my))
        cp_r[0].wait()
        cp_r[1].wait()
        cp_g.wait()

        out_dma = []

        def store_out(off):
            d = pltpu.make_async_copy(
                ag_ref.at[pl.ds(off, hs), :],
                out_hbm.at[pl.ds(off, hs), :],
                lsem.at[10 + len(out_dma)],
            )
            d.start()
            out_dma.append(d)

        def norm_half(off):
            s = r1_ref[pl.ds(off, hs), :] + r2_ref[pl.ds(off, hs), :]
            y = s.astype(jnp.float32) + rv_ref[pl.ds(off, hs), :]
            ms = jnp.mean(y * y, axis=-1, keepdims=True)
            o = y * lax.rsqrt(ms + 1e-6) * gv_ref[...]
            ag_ref[pl.ds(off, hs), :] = o.astype(jnp.bfloat16)

        for c in (0, 1):
            t[4 + c].wait_recv()
            norm_half(rowA(my) + c * hs)
            t[8 + c] = make(ag_ref, ag_ref, rowA(my) + c * hs, hs, p2, 8 + c)
            t[8 + c].start()
            store_out(rowA(my) + c * hs)
        for c in (0, 1):
            t[6 + c].wait_recv()
            norm_half(rowB(my) + c * hs)
            t[10 + c] = make(ag_ref, ag_ref, rowB(my) + c * hs, hs, p1, 10 + c)
            t[10 + c].start()
            store_out(rowB(my) + c * hs)
        for c in (0, 1):
            t[12 + c] = make(ag_ref, ag_ref, rowB(my) + c * hs, hs, p2, 12 + c)
            t[12 + c].start()
        for c in (0, 1):
            t[14 + c] = make(ag_ref, ag_ref, rowA(my) + c * hs, hs, p1, 14 + c)
            t[14 + c].start()

        for c in (0, 1):
            t[8 + c].wait_recv()
            t[16 + c] = make(ag_ref, ag_ref, rowA(p2) + c * hs, hs, p1, 16 + c)
            t[16 + c].start()
            store_out(rowA(p2) + c * hs)
        for c in (0, 1):
            t[10 + c].wait_recv()
            t[18 + c] = make(ag_ref, ag_ref, rowB(p1) + c * hs, hs, p2, 18 + c)
            t[18 + c].start()
            store_out(rowB(p1) + c * hs)

        for i, off in ((12, rowB(p2)), (13, rowB(p2) + hs),
                       (14, rowA(p1)), (15, rowA(p1) + hs),
                       (16, rowA(p2 ^ 1)), (17, rowA(p2 ^ 1) + hs),
                       (18, rowB(p2 ^ 1)), (19, rowB(p2 ^ 1) + hs)):
            t[i].wait_recv()
            store_out(off)

        for d in out_dma:
            d.wait()
        for i in range(20):
            t[i].wait_send()

    return pl.pallas_call(
        body,
        out_shape=jax.ShapeDtypeStruct((m, n), jnp.bfloat16),
        in_specs=[
            pl.BlockSpec(memory_space=pltpu.MemorySpace.HBM),
            pl.BlockSpec(memory_space=pltpu.MemorySpace.HBM),
            pl.BlockSpec(memory_space=pltpu.MemorySpace.HBM),
        ],
        out_specs=pl.BlockSpec(memory_space=pltpu.MemorySpace.HBM),
        scratch_shapes=[
            pltpu.VMEM((m, n), jnp.float32),
            pltpu.VMEM((m, n), jnp.float32),
            pltpu.VMEM((1, n), jnp.float32),
            pltpu.VMEM((m, n), jnp.bfloat16),
            pltpu.VMEM((m, n), jnp.bfloat16),
            pltpu.VMEM((m, n), jnp.bfloat16),
            pltpu.VMEM((m, n), jnp.bfloat16),
            pltpu.SemaphoreType.DMA((20,)),
            pltpu.SemaphoreType.DMA((20,)),
            pltpu.SemaphoreType.DMA((27,)),
        ],
        compiler_params=pltpu.CompilerParams(collective_id=0),
    )(x, resid, gamma2d)


# device time: 26106 ns/iter; 1.0273x vs baseline; 1.0273x over previous
import jax
import jax.numpy as jnp
from jax import lax
from jax.experimental import pallas as pl
from jax.experimental.pallas import tpu as pltpu

N_DEV = 4


def kernel(partial, resid, gamma):
    x = partial.reshape(partial.shape[-2], partial.shape[-1])
    m, n = x.shape
    bs = m // 8
    gamma2d = gamma.reshape(1, n)
    x = pltpu.with_memory_space_constraint(x, pltpu.MemorySpace.HBM)
    resid = pltpu.with_memory_space_constraint(resid, pltpu.MemorySpace.HBM)
    gamma2d = pltpu.with_memory_space_constraint(gamma2d, pltpu.MemorySpace.HBM)

    def body(x_hbm, resid_hbm, gamma_hbm, out_hbm,
             xv_ref, rv_ref, gv_ref,
             xb_ref, r1_ref, r2_ref, ag_ref,
             send_sems, recv_sems, lsem):
        my = lax.axis_index("i")
        p1 = my ^ 1
        p2 = 3 - my

        def rowA(j):
            return j * bs

        def rowB(j):
            return (4 + j) * bs

        send_offs = (rowA(p2 ^ 1), rowB(p2 ^ 1), rowA(p1), rowB(p2))
        held_offs = (rowA(p2), rowB(p1), rowA(my), rowB(my))
        dx = []
        for k, off in enumerate(send_offs + held_offs):
            d = pltpu.make_async_copy(
                x_hbm.at[pl.ds(off, bs), :],
                xv_ref.at[pl.ds(off, bs), :],
                lsem.at[k],
            )
            d.start()
            dx.append(d)
        cp_r = []
        for k, off in enumerate((rowA(my), rowB(my))):
            d = pltpu.make_async_copy(
                resid_hbm.at[pl.ds(off, bs), :],
                rv_ref.at[pl.ds(off, bs), :],
                lsem.at[8 + k],
            )
            d.start()
            cp_r.append(d)
        cp_g = pltpu.make_async_copy(gamma_hbm, gv_ref, lsem.at[18])
        cp_g.start()

        barrier_sem = pltpu.get_barrier_semaphore()
        for nbr in (p1, p2):
            pl.semaphore_signal(
                barrier_sem, inc=1,
                device_id=(nbr,), device_id_type=pl.DeviceIdType.MESH,
            )
        pl.semaphore_wait(barrier_sem, 2)

        def make(src_ref, dst_ref, off, partner, i):
            return pltpu.make_async_remote_copy(
                src_ref=src_ref.at[pl.ds(off, bs), :],
                dst_ref=dst_ref.at[pl.ds(off, bs), :],
                send_sem=send_sems.at[i],
                recv_sem=recv_sems.at[i],
                device_id=(partner,),
                device_id_type=pl.DeviceIdType.MESH,
            )

        def acc(off):
            r1_ref[pl.ds(off, bs), :] = (
                r1_ref[pl.ds(off, bs), :]
                + xv_ref[pl.ds(off, bs), :].astype(jnp.bfloat16)
            )

        t = {}

        def cast_send(dma, off, tgt, i):
            dma.wait()
            xb_ref[pl.ds(off, bs), :] = (
                xv_ref[pl.ds(off, bs), :].astype(jnp.bfloat16)
            )
            t[i] = make(xb_ref, r1_ref, off, tgt, i)
            t[i].start()

        cast_send(dx[0], rowA(p2 ^ 1), p1, 0)
        cast_send(dx[1], rowB(p2 ^ 1), p2, 2)
        cast_send(dx[2], rowA(p1), p1, 1)
        cast_send(dx[3], rowB(p2), p2, 3)

        t[0].wait_recv()
        dx[4].wait()
        acc(rowA(p2))
        t[4] = make(r1_ref, r2_ref, rowA(p2), p2, 4)
        t[4].start()

        t[2].wait_recv()
        dx[5].wait()
        acc(rowB(p1))
        t[5] = make(r1_ref, r2_ref, rowB(p1), p1, 5)
        t[5].start()

        t[1].wait_recv()
        dx[6].wait()
        acc(rowA(my))
        t[3].wait_recv()
        dx[7].wait()
        acc(rowB(my))
        cp_r[0].wait()
        cp_r[1].wait()
        cp_g.wait()

        out_dma = []

        def store_out(off):
            d = pltpu.make_async_copy(
                ag_ref.at[pl.ds(off, bs), :],
                out_hbm.at[pl.ds(off, bs), :],
                lsem.at[10 + len(out_dma)],
            )
            d.start()
            out_dma.append(d)

        def norm_block(off):
            s = r1_ref[pl.ds(off, bs), :] + r2_ref[pl.ds(off, bs), :]
            y = s.astype(jnp.float32) + rv_ref[pl.ds(off, bs), :]
            ms = jnp.mean(y * y, axis=-1, keepdims=True)
            o = y * lax.rsqrt(ms + 1e-6) * gv_ref[...]
            ag_ref[pl.ds(off, bs), :] = o.astype(jnp.bfloat16)

        hs = bs // 2

        def make_h(off, partner, i):
            return pltpu.make_async_remote_copy(
                src_ref=ag_ref.at[pl.ds(off, hs), :],
                dst_ref=ag_ref.at[pl.ds(off, hs), :],
                send_sem=send_sems.at[i],
                recv_sem=recv_sems.at[i],
                device_id=(partner,),
                device_id_type=pl.DeviceIdType.MESH,
            )

        t[4].wait_recv()
        norm_block(rowA(my))
        t[6] = make_h(rowA(my), p2, 6)
        t[7] = make_h(rowA(my) + hs, p2, 7)
        t[6].start()
        t[7].start()
        store_out(rowA(my))

        t[5].wait_recv()
        norm_block(rowB(my))
        t[8] = make_h(rowB(my), p1, 8)
        t[9] = make_h(rowB(my) + hs, p1, 9)
        t[10] = make(ag_ref, ag_ref, rowB(my), p2, 10)
        t[11] = make(ag_ref, ag_ref, rowA(my), p1, 11)
        t[8].start()
        t[9].start()
        t[10].start()
        t[11].start()
        store_out(rowB(my))

        t[6].wait_recv()
        t[12] = make_h(rowA(p2), p1, 12)
        t[12].start()
        t[7].wait_recv()
        t[13] = make_h(rowA(p2) + hs, p1, 13)
        t[13].start()
        store_out(rowA(p2))

        t[8].wait_recv()
        t[14] = make_h(rowB(p1), p2, 14)
        t[14].start()
        t[9].wait_recv()
        t[15] = make_h(rowB(p1) + hs, p2, 15)
        t[15].start()
        store_out(rowB(p1))

        t[10].wait_recv()
        store_out(rowB(p2))
        t[11].wait_recv()
        store_out(rowA(p1))
        t[12].wait_recv()
        t[13].wait_recv()
        store_out(rowA(p2 ^ 1))
        t[14].wait_recv()
        t[15].wait_recv()
        store_out(rowB(p2 ^ 1))

        for d in out_dma:
            d.wait()
        for i in range(16):
            t[i].wait_send()

    return pl.pallas_call(
        body,
        out_shape=jax.ShapeDtypeStruct((m, n), jnp.bfloat16),
        in_specs=[
            pl.BlockSpec(memory_space=pltpu.MemorySpace.HBM),
            pl.BlockSpec(memory_space=pltpu.MemorySpace.HBM),
            pl.BlockSpec(memory_space=pltpu.MemorySpace.HBM),
        ],
        out_specs=pl.BlockSpec(memory_space=pltpu.MemorySpace.HBM),
        scratch_shapes=[
            pltpu.VMEM((m, n), jnp.float32),
            pltpu.VMEM((m, n), jnp.float32),
            pltpu.VMEM((1, n), jnp.float32),
            pltpu.VMEM((m, n), jnp.bfloat16),
            pltpu.VMEM((m, n), jnp.bfloat16),
            pltpu.VMEM((m, n), jnp.bfloat16),
            pltpu.VMEM((m, n), jnp.bfloat16),
            pltpu.SemaphoreType.DMA((16,)),
            pltpu.SemaphoreType.DMA((16,)),
            pltpu.SemaphoreType.DMA((19,)),
        ],
        compiler_params=pltpu.CompilerParams(collective_id=0),
    )(x, resid, gamma2d)
